# Initial kernel scaffold; baseline (speedup 1.0000x reference)
import jax
import jax.numpy as jnp
from jax.experimental import pallas as pl


def kernel(input):
    raise NotImplementedError("write your pallas kernel here")


# speedup vs baseline: 5.9048x; 5.9048x over previous
import jax
import jax.numpy as jnp
from jax.experimental import pallas as pl
from jax.experimental.pallas import tpu as pltpu

ROWS = 128
COLS = 32768
K = 8
BLK = 4096
NBLK = COLS // BLK
SUB = BLK // K
ACCW = 128

_NET8 = [
    (0, 1), (2, 3), (4, 5), (6, 7),
    (0, 2), (1, 3), (4, 6), (5, 7),
    (1, 2), (5, 6),
    (0, 4), (1, 5), (2, 6), (3, 7),
    (2, 4), (3, 5),
    (1, 2), (3, 4), (5, 6),
]

_BITONIC8 = [
    (0, 4), (1, 5), (2, 6), (3, 7),
    (0, 2), (1, 3), (4, 6), (5, 7),
    (0, 1), (2, 3), (4, 5), (6, 7),
]


def _apply_net(vs, net):
    vs = list(vs)
    for i, j in net:
        a, b = vs[i], vs[j]
        vs[i] = jnp.maximum(a, b)
        vs[j] = jnp.minimum(a, b)
    return vs


def _merge_top8(avs, bvs):
    c = [jnp.maximum(avs[i], bvs[K - 1 - i]) for i in range(K)]
    return _apply_net(c, _BITONIC8)


def _halve(vs):
    w = vs[0].shape[1] // 2
    a = [v[:, :w] for v in vs]
    b = [v[:, w:] for v in vs]
    return _merge_top8(a, b)


def _topk_kernel(x_ref, o_ref, acc_ref):
    step = pl.program_id(0)
    vs = [x_ref[:, i * SUB:(i + 1) * SUB] for i in range(K)]
    vs = _apply_net(vs, _NET8)
    while vs[0].shape[1] > ACCW:
        vs = _halve(vs)

    @pl.when(step == 0)
    def _init():
        acc_ref[...] = jnp.concatenate(vs, axis=1)

    @pl.when(step != 0)
    def _accumulate():
        accv = [acc_ref[:, i * ACCW:(i + 1) * ACCW] for i in range(K)]
        merged = _merge_top8(accv, vs)
        acc_ref[...] = jnp.concatenate(merged, axis=1)

    @pl.when(step == NBLK - 1)
    def _finalize():
        accv = [acc_ref[:, i * ACCW:(i + 1) * ACCW] for i in range(K)]
        while accv[0].shape[1] > 1:
            accv = _halve(accv)
        o_ref[...] = jnp.concatenate(accv, axis=1)


def kernel(input):
    return pl.pallas_call(
        _topk_kernel,
        grid=(NBLK,),
        in_specs=[pl.BlockSpec((ROWS, BLK), lambda i: (0, i))],
        out_specs=pl.BlockSpec((ROWS, K), lambda i: (0, 0)),
        out_shape=jax.ShapeDtypeStruct((ROWS, K), jnp.float32),
        scratch_shapes=[pltpu.VMEM((ROWS, K * ACCW), jnp.float32)],
    )(input)
